# Initial kernel scaffold; baseline (speedup 1.0000x reference)
import functools

import jax
import jax.numpy as jnp
from jax import lax
from jax.experimental import pallas as pl
from jax.experimental.pallas import tpu as pltpu
from jax.experimental.pallas import tpu_sc as plsc

N = 10000
E = 320000
D = 128
F = 128
NEG = 0.2

NC = 2
NS = 16
NW = NC * NS
EW = E // NW
BB = 80
NB = EW // BB
VI = EW // 16
NI = N // 16
NROWS = N // NS

_mesh = plsc.VectorSubcoreMesh(core_axis_name="c", subcore_axis_name="s")


def _dense_body(x_ref, ea_ref, wfc_ref, wres_ref, we_ref, al_ref, ar_ref,
                ae_ref, feat_ref, res_ref, el_ref, er_ref, ee_ref, c_ref):
    x = x_ref[...]
    feat = jnp.dot(x, wfc_ref[...], preferred_element_type=jnp.float32)
    feat_ref[...] = feat
    res_ref[...] = jnp.dot(x, wres_ref[...], preferred_element_type=jnp.float32)
    el = jnp.sum(feat * al_ref[...][None, :], axis=1)
    er = jnp.sum(feat * ar_ref[...][None, :], axis=1)
    el_ref[...] = el
    er_ref[...] = er
    wv = jnp.sum(we_ref[...] * ae_ref[...][None, :], axis=1)
    ee = jnp.sum(ea_ref[...] * wv[None, :], axis=1)
    ee_ref[...] = ee
    craw = jnp.max(el) + jnp.max(er) + jnp.max(ee)
    cval = jnp.where(craw >= 0, craw, NEG * craw)
    c_ref[...] = jnp.full((16,), cval, jnp.float32)


def _dense(x, ea, wfc, wres, we, al, ar, ae):
    return pl.pallas_call(
        _dense_body,
        out_shape=(
            jax.ShapeDtypeStruct((N, F), jnp.float32),
            jax.ShapeDtypeStruct((N, F), jnp.float32),
            jax.ShapeDtypeStruct((N,), jnp.float32),
            jax.ShapeDtypeStruct((N,), jnp.float32),
            jax.ShapeDtypeStruct((E,), jnp.float32),
            jax.ShapeDtypeStruct((16,), jnp.float32),
        ),
    )(x, ea, wfc, wres, we, al, ar, ae)


@functools.partial(
    pl.kernel,
    out_type=(
        jax.ShapeDtypeStruct((E,), jnp.float32),
        jax.ShapeDtypeStruct((2, N), jnp.float32),
        jax.ShapeDtypeStruct((2, N), jnp.float32),
    ),
    mesh=_mesh,
    scratch_types=[
        pltpu.VMEM((EW,), jnp.int32),
        pltpu.VMEM((EW,), jnp.int32),
        pltpu.VMEM((EW,), jnp.float32),
        pltpu.VMEM((EW,), jnp.float32),
        pltpu.VMEM((N,), jnp.float32),
        pltpu.VMEM((N,), jnp.float32),
        pltpu.VMEM((N,), jnp.float32),
        pltpu.VMEM((N,), jnp.float32),
        pltpu.VMEM((16,), jnp.float32),
        pltpu.VMEM_SHARED((N,), jnp.float32),
        pltpu.VMEM_SHARED((N,), jnp.float32),
    ],
)
def _phase1(src_h, dst_h, el_h, er_h, ee_h, c_h, p_h, esum_h, deg_h,
            src_c, dst_c, ee_c, p_c, el_t, er_t, esum_l, deg_l, cv,
            esum_s, deg_s):
    c = lax.axis_index("c")
    s = lax.axis_index("s")
    wid = s * NC + c
    base = wid * EW
    pltpu.sync_copy(src_h.at[pl.ds(base, EW)], src_c)
    pltpu.sync_copy(dst_h.at[pl.ds(base, EW)], dst_c)
    pltpu.sync_copy(ee_h.at[pl.ds(base, EW)], ee_c)
    pltpu.sync_copy(el_h, el_t)
    pltpu.sync_copy(er_h, er_t)
    pltpu.sync_copy(c_h, cv)

    zero = jnp.zeros((16,), jnp.float32)

    @pl.loop(0, NI)
    def _(i):
        sl = pl.ds(i * 16, 16)
        esum_l[sl] = zero
        deg_l[sl] = zero

    @pl.when(s == 0)
    def _():
        pltpu.sync_copy(esum_l, esum_s)
        pltpu.sync_copy(deg_l, deg_s)

    plsc.subcore_barrier()

    cval = cv[...]
    ones = jnp.ones((16,), jnp.float32)

    @pl.loop(0, VI)
    def _(i):
        sl = pl.ds(i * 16, 16)
        s16 = src_c[sl]
        d16 = dst_c[sl]
        elv = plsc.load_gather(el_t, [s16])
        erv = plsc.load_gather(er_t, [d16])
        raw = elv + erv + ee_c[sl]
        e = jnp.where(raw >= 0, raw, NEG * raw)
        pv = jnp.exp(e - cval)
        p_c[sl] = pv
        plsc.addupdate_scatter(esum_l, [d16], pv)
        plsc.addupdate_scatter(deg_l, [d16], ones)

    pltpu.sync_copy(p_c, p_h.at[pl.ds(base, EW)])
    pltpu.sync_copy(esum_l, esum_s, add=True)
    pltpu.sync_copy(deg_l, deg_s, add=True)
    plsc.subcore_barrier()

    @pl.when(s == 0)
    def _():
        pltpu.sync_copy(esum_s, esum_h.at[c])
        pltpu.sync_copy(deg_s, deg_h.at[c])


@functools.partial(
    pl.kernel,
    out_type=jax.ShapeDtypeStruct((2, N, F), jnp.float32),
    mesh=_mesh,
    scratch_types=[
        pltpu.VMEM((NB, BB), jnp.int32),
        pltpu.VMEM((NB, BB), jnp.int32),
        pltpu.VMEM((NB, BB), jnp.float32),
        pltpu.VMEM((N,), jnp.float32),
        pltpu.VMEM((N,), jnp.float32),
        pltpu.VMEM((BB,), jnp.float32),
        pltpu.VMEM((BB, F), jnp.float32),
        pltpu.VMEM_SHARED((N, F), jnp.float32),
        pltpu.SemaphoreType.DMA,
    ],
)
def _phase2(src_h, dst_h, p_h, esum_h, feat_h, rst_h,
            src_c, dst_c, p_c, es_t, tmp_t, a_buf, rows, rst_s, sem):
    c = lax.axis_index("c")
    s = lax.axis_index("s")
    wid = s * NC + c
    rbase = wid * NB
    pltpu.sync_copy(src_h.at[pl.ds(rbase, NB)], src_c)
    pltpu.sync_copy(dst_h.at[pl.ds(rbase, NB)], dst_c)
    pltpu.sync_copy(p_h.at[pl.ds(rbase, NB)], p_c)
    pltpu.sync_copy(esum_h.at[0], es_t)
    pltpu.sync_copy(esum_h.at[1], tmp_t)

    @pl.loop(0, NI)
    def _(i):
        sl = pl.ds(i * 16, 16)
        es_t[sl] = es_t[sl] + tmp_t[sl]

    zero = jnp.zeros((16,), jnp.float32)

    @pl.loop(0, BB)
    def _(r):
        for j in range(F // 16):
            rows[r, pl.ds(j * 16, 16)] = zero

    @pl.loop(0, NROWS // BB)
    def _(k):
        pltpu.sync_copy(rows, rst_s.at[pl.ds(s * NROWS + k * BB, BB), :])

    rem = NROWS % BB
    if rem:
        pltpu.sync_copy(
            rows.at[pl.ds(0, rem)],
            rst_s.at[pl.ds(s * NROWS + (NROWS // BB) * BB, rem), :])

    plsc.subcore_barrier()

    @pl.loop(0, NB)
    def _(b):
        cp = pltpu.async_copy(feat_h.at[src_c.at[b]], rows, sem)

        @pl.loop(0, BB // 16)
        def _(j):
            sl = pl.ds(j * 16, 16)
            d16 = dst_c[b, sl]
            ev = plsc.load_gather(es_t, [d16])
            a_buf[sl] = p_c[b, sl] / ev

        cp.wait()

        @pl.loop(0, BB)
        def _(r):
            av = a_buf[r]
            for j in range(F // 16):
                sl = pl.ds(j * 16, 16)
                rows[r, sl] = rows[r, sl] * av

        pltpu.sync_copy(rows, rst_s.at[dst_c.at[b]], add=True)

    plsc.subcore_barrier()
    pltpu.sync_copy(rst_s.at[pl.ds(s * NROWS, NROWS), :],
                    rst_h.at[c, pl.ds(s * NROWS, NROWS), :])


def _final_body(rst_ref, deg_ref, res_ref, o_ref):
    deg = deg_ref[0] + deg_ref[1]
    norm = 1.0 / jnp.maximum(deg, 1.0)
    o_ref[...] = (rst_ref[0] + rst_ref[1]) * norm[:, None] + res_ref[...]


def _final(rst_part, deg_part, resval):
    return pl.pallas_call(
        _final_body,
        out_shape=jax.ShapeDtypeStruct((N, F), jnp.float32),
    )(rst_part, deg_part, resval)


def kernel(x, edge_index, edge_attr, W_fc, W_edge, attn_l, attn_r, attn_edge,
           W_res):
    src = edge_index[0]
    dst = edge_index[1]
    al = attn_l.reshape(F)
    ar = attn_r.reshape(F)
    ae = attn_edge.reshape(F)
    feat, resval, el, er, ee, cvec = _dense(
        x, edge_attr, W_fc, W_res, W_edge, al, ar, ae)
    p, esum_part, deg_part = _phase1(src, dst, el, er, ee, cvec)
    src2 = src.reshape(E // BB, BB)
    dst2 = dst.reshape(E // BB, BB)
    p2 = p.reshape(E // BB, BB)
    rst_part = _phase2(src2, dst2, p2, esum_part, feat)
    out = _final(rst_part, deg_part, resval)
    return out.reshape(N, 1, F)


# speedup vs baseline: 25.0161x; 25.0161x over previous
import functools

import jax
import jax.numpy as jnp
from jax import lax
from jax.experimental import pallas as pl
from jax.experimental.pallas import tpu as pltpu
from jax.experimental.pallas import tpu_sc as plsc

N = 10000
E = 320000
D = 128
F = 128
NEG = 0.2

NC = 2
NS = 16
NW = NC * NS
EW = E // NW
BB = 80
NB = EW // BB
VI = EW // 16
NP = 10240
NPI = NP // 16
RT = NP // NS

_mesh = plsc.VectorSubcoreMesh(core_axis_name="c", subcore_axis_name="s")
_sc_params = pltpu.CompilerParams(needs_layout_passes=False)


def _escore_body(ea_ref, we_ref, ae_ref, ee_ref, eemax_ref):
    wv = jnp.sum(we_ref[...] * ae_ref[...][None, :], axis=1)
    ee = jnp.sum(ea_ref[...] * wv[:, None], axis=0)
    ee_ref[...] = ee
    eemax_ref[...] = jnp.full((16,), jnp.max(ee), jnp.float32)


def _escore(ea, we, ae):
    return pl.pallas_call(
        _escore_body,
        out_shape=(
            jax.ShapeDtypeStruct((E,), jnp.float32),
            jax.ShapeDtypeStruct((16,), jnp.float32),
        ),
    )(ea, we, ae)


def _dense_body(x_ref, wfc_ref, wres_ref, al_ref, ar_ref, eemax_ref,
                feat_ref, res_ref, el_ref, er_ref, c_ref):
    x = x_ref[...]
    feat = jnp.dot(x, wfc_ref[...], preferred_element_type=jnp.float32)
    feat_ref[...] = feat
    res_ref[...] = jnp.dot(x, wres_ref[...], preferred_element_type=jnp.float32)
    el = jnp.sum(feat * al_ref[...][None, :], axis=1)
    er = jnp.sum(feat * ar_ref[...][None, :], axis=1)
    el_ref[...] = el
    er_ref[...] = er
    craw = jnp.max(el) + jnp.max(er) + eemax_ref[0]
    cval = jnp.where(craw >= 0, craw, NEG * craw)
    c_ref[...] = jnp.full((16,), cval, jnp.float32)


def _dense(x, wfc, wres, al, ar, eemax):
    return pl.pallas_call(
        _dense_body,
        out_shape=(
            jax.ShapeDtypeStruct((N, F), jnp.float32),
            jax.ShapeDtypeStruct((N, F), jnp.float32),
            jax.ShapeDtypeStruct((N,), jnp.float32),
            jax.ShapeDtypeStruct((N,), jnp.float32),
            jax.ShapeDtypeStruct((16,), jnp.float32),
        ),
    )(x, wfc, wres, al, ar, eemax)


@functools.partial(
    pl.kernel,
    out_type=(
        jax.ShapeDtypeStruct((E,), jnp.float32),
        jax.ShapeDtypeStruct((NW * NP,), jnp.float32),
        jax.ShapeDtypeStruct((NW * NP,), jnp.float32),
    ),
    mesh=_mesh,
    compiler_params=_sc_params,
    scratch_types=[
        pltpu.VMEM((EW,), jnp.int32),
        pltpu.VMEM((EW,), jnp.int32),
        pltpu.VMEM((EW,), jnp.float32),
        pltpu.VMEM((EW,), jnp.float32),
        pltpu.VMEM((N,), jnp.float32),
        pltpu.VMEM((N,), jnp.float32),
        pltpu.VMEM((NP,), jnp.float32),
        pltpu.VMEM((NP,), jnp.float32),
        pltpu.VMEM((16,), jnp.float32),
    ],
)
def _phase1(src_h, dst_h, el_h, er_h, ee_h, c_h, p_h, esum_h, deg_h,
            src_c, dst_c, ee_c, p_c, el_t, er_t, esum_l, deg_l, cv):
    c = lax.axis_index("c")
    s = lax.axis_index("s")
    wid = s * NC + c
    base = wid * EW
    pltpu.sync_copy(src_h.at[pl.ds(base, EW)], src_c)
    pltpu.sync_copy(dst_h.at[pl.ds(base, EW)], dst_c)
    pltpu.sync_copy(ee_h.at[pl.ds(base, EW)], ee_c)
    pltpu.sync_copy(el_h, el_t)
    pltpu.sync_copy(er_h, er_t)
    pltpu.sync_copy(c_h, cv)

    zero = jnp.zeros((16,), jnp.float32)

    @pl.loop(0, NPI)
    def _(i):
        sl = pl.ds(i * 16, 16)
        esum_l[sl] = zero
        deg_l[sl] = zero

    cval = cv[...]
    ones = jnp.ones((16,), jnp.float32)

    @pl.loop(0, VI)
    def _(i):
        sl = pl.ds(i * 16, 16)
        s16 = src_c[sl]
        d16 = dst_c[sl]
        elv = plsc.load_gather(el_t, [s16])
        erv = plsc.load_gather(er_t, [d16])
        raw = elv + erv + ee_c[sl]
        e = jnp.where(raw >= 0, raw, NEG * raw)
        pv = jnp.exp(e - cval)
        p_c[sl] = pv
        plsc.addupdate_scatter(esum_l, [d16], pv)
        plsc.addupdate_scatter(deg_l, [d16], ones)

    pltpu.sync_copy(p_c, p_h.at[pl.ds(base, EW)])
    pltpu.sync_copy(esum_l, esum_h.at[pl.ds(wid * NP, NP)])
    pltpu.sync_copy(deg_l, deg_h.at[pl.ds(wid * NP, NP)])


def _mid_body(esum32_ref, deg32_ref, esum_ref, norm_ref):
    esum_ref[...] = jnp.sum(esum32_ref[...], axis=0)
    deg = jnp.sum(deg32_ref[...], axis=0)
    norm_ref[...] = 1.0 / jnp.maximum(deg, 1.0)


def _mid(esum32, deg32):
    return pl.pallas_call(
        _mid_body,
        out_shape=(
            jax.ShapeDtypeStruct((NP,), jnp.float32),
            jax.ShapeDtypeStruct((NP,), jnp.float32),
        ),
    )(esum32, deg32)


@functools.partial(
    pl.kernel,
    out_type=jax.ShapeDtypeStruct((2, NP, F), jnp.float32),
    mesh=_mesh,
    compiler_params=_sc_params,
    scratch_types=[
        pltpu.VMEM((BB,), jnp.int32),
        pltpu.VMEM((BB,), jnp.int32),
        pltpu.VMEM((EW,), jnp.float32),
        pltpu.VMEM((NP,), jnp.float32),
        pltpu.VMEM((BB,), jnp.float32),
        pltpu.VMEM((BB, F), jnp.float32),
        pltpu.VMEM_SHARED((NP, F), jnp.float32),
        pltpu.SemaphoreType.DMA,
    ],
)
def _phase2(src_h, dst_h, p_h, esum_h, feat_h, rst_h,
            sidx, didx, p_c, es_t, a_buf, rows, rst_s, sem):
    c = lax.axis_index("c")
    s = lax.axis_index("s")
    wid = s * NC + c
    base = wid * EW
    pltpu.sync_copy(p_h.at[pl.ds(base, EW)], p_c)
    pltpu.sync_copy(esum_h, es_t)

    zero = jnp.zeros((16,), jnp.float32)

    @pl.loop(0, BB)
    def _(r):
        for j in range(F // 16):
            rows[r, pl.ds(j * 16, 16)] = zero

    @pl.loop(0, RT // BB)
    def _(k):
        pltpu.sync_copy(rows, rst_s.at[pl.ds(s * RT + k * BB, BB), :])

    plsc.subcore_barrier()

    @pl.loop(0, NB)
    def _(b):
        ebase = b * BB
        pltpu.sync_copy(src_h.at[pl.ds(base + ebase, BB)], sidx)
        pltpu.sync_copy(dst_h.at[pl.ds(base + ebase, BB)], didx)

        cp = pltpu.async_copy(feat_h.at[sidx], rows, sem)

        @pl.loop(0, BB // 16)
        def _(j):
            sl = pl.ds(j * 16, 16)
            d16 = didx[sl]
            ev = plsc.load_gather(es_t, [d16])
            a_buf[sl] = p_c[pl.ds(ebase + j * 16, 16)] / ev

        cp.wait()

        @pl.loop(0, BB // 16)
        def _(g):
            a16 = a_buf[pl.ds(g * 16, 16)]
            for r in range(16):
                e = g * 16 + r
                av = a16[r]
                for j in range(F // 16):
                    sl = pl.ds(j * 16, 16)
                    rows[e, sl] = rows[e, sl] * av

        pltpu.sync_copy(rows, rst_s.at[didx], add=True)

    plsc.subcore_barrier()
    pltpu.sync_copy(rst_s.at[pl.ds(s * RT, RT), :],
                    rst_h.at[c, pl.ds(s * RT, RT), :])


def _final_body(rst_ref, norm_ref, res_ref, o_ref):
    norm = norm_ref[...][:N]
    rst = rst_ref[0, :N, :] + rst_ref[1, :N, :]
    o_ref[...] = rst * norm[:, None] + res_ref[...]


def _final(rst_part, norm, resval):
    return pl.pallas_call(
        _final_body,
        out_shape=jax.ShapeDtypeStruct((N, F), jnp.float32),
    )(rst_part, norm, resval)


def kernel(x, edge_index, edge_attr, W_fc, W_edge, attn_l, attn_r, attn_edge,
           W_res):
    src = edge_index[0]
    dst = edge_index[1]
    al = attn_l.reshape(F)
    ar = attn_r.reshape(F)
    ae = attn_edge.reshape(F)
    ee, eemax = _escore(edge_attr.T, W_edge, ae)
    feat, resval, el, er, cvec = _dense(x, W_fc, W_res, al, ar, eemax)
    p, esum32, deg32 = _phase1(src, dst, el, er, ee, cvec)
    esum_f, norm_f = _mid(esum32.reshape(NW, NP), deg32.reshape(NW, NP))
    rst_part = _phase2(src, dst, p, esum_f, feat)
    out = _final(rst_part, norm_f, resval)
    return out.reshape(N, 1, F)


# speedup vs baseline: 41.1209x; 1.6438x over previous
import functools

import jax
import jax.numpy as jnp
from jax import lax
from jax.experimental import pallas as pl
from jax.experimental.pallas import tpu as pltpu
from jax.experimental.pallas import tpu_sc as plsc

N = 10000
E = 320000
D = 128
F = 128
NEG = 0.2

NC = 2
NS = 16
NW = NC * NS
EW = E // NW
BB = 80
NB = EW // BB
VI = EW // 16
NP = 10240
NPI = NP // 16
RT = NP // NS

_mesh = plsc.VectorSubcoreMesh(core_axis_name="c", subcore_axis_name="s")
_sc_params = pltpu.CompilerParams(needs_layout_passes=False)


def _escore_body(ea_ref, we_ref, ae_ref, ee_ref, eemax_ref):
    wv = jnp.sum(we_ref[...] * ae_ref[...][None, :], axis=1)
    ee = jnp.sum(ea_ref[...] * wv[:, None], axis=0)
    ee_ref[...] = ee
    eemax_ref[...] = jnp.full((16,), jnp.max(ee), jnp.float32)


def _escore(ea, we, ae):
    return pl.pallas_call(
        _escore_body,
        out_shape=(
            jax.ShapeDtypeStruct((E,), jnp.float32),
            jax.ShapeDtypeStruct((16,), jnp.float32),
        ),
    )(ea, we, ae)


def _dense_body(x_ref, wfc_ref, wres_ref, al_ref, ar_ref, eemax_ref,
                feat_ref, res_ref, el_ref, er_ref, c_ref):
    x = x_ref[...]
    feat = jnp.dot(x, wfc_ref[...], preferred_element_type=jnp.float32)
    feat_ref[...] = feat
    res_ref[...] = jnp.dot(x, wres_ref[...], preferred_element_type=jnp.float32)
    el = jnp.sum(feat * al_ref[...][None, :], axis=1)
    er = jnp.sum(feat * ar_ref[...][None, :], axis=1)
    el_ref[...] = el
    er_ref[...] = er
    craw = jnp.max(el) + jnp.max(er) + eemax_ref[0]
    cval = jnp.where(craw >= 0, craw, NEG * craw)
    c_ref[...] = jnp.full((16,), cval, jnp.float32)


def _dense(x, wfc, wres, al, ar, eemax):
    return pl.pallas_call(
        _dense_body,
        out_shape=(
            jax.ShapeDtypeStruct((N, F), jnp.float32),
            jax.ShapeDtypeStruct((N, F), jnp.float32),
            jax.ShapeDtypeStruct((N,), jnp.float32),
            jax.ShapeDtypeStruct((N,), jnp.float32),
            jax.ShapeDtypeStruct((16,), jnp.float32),
        ),
    )(x, wfc, wres, al, ar, eemax)


@functools.partial(
    pl.kernel,
    out_type=(
        jax.ShapeDtypeStruct((E,), jnp.float32),
        jax.ShapeDtypeStruct((NW * NP,), jnp.float32),
        jax.ShapeDtypeStruct((NW * NP,), jnp.float32),
    ),
    mesh=_mesh,
    compiler_params=_sc_params,
    scratch_types=[
        pltpu.VMEM((EW,), jnp.int32),
        pltpu.VMEM((EW,), jnp.int32),
        pltpu.VMEM((EW,), jnp.float32),
        pltpu.VMEM((EW,), jnp.float32),
        pltpu.VMEM((N,), jnp.float32),
        pltpu.VMEM((N,), jnp.float32),
        pltpu.VMEM((NP,), jnp.float32),
        pltpu.VMEM((NP,), jnp.float32),
        pltpu.VMEM((16,), jnp.float32),
    ],
)
def _phase1(src_h, dst_h, el_h, er_h, ee_h, c_h, p_h, esum_h, deg_h,
            src_c, dst_c, ee_c, p_c, el_t, er_t, esum_l, deg_l, cv):
    c = lax.axis_index("c")
    s = lax.axis_index("s")
    wid = s * NC + c
    base = wid * EW
    pltpu.sync_copy(src_h.at[pl.ds(base, EW)], src_c)
    pltpu.sync_copy(dst_h.at[pl.ds(base, EW)], dst_c)
    pltpu.sync_copy(ee_h.at[pl.ds(base, EW)], ee_c)
    pltpu.sync_copy(el_h, el_t)
    pltpu.sync_copy(er_h, er_t)
    pltpu.sync_copy(c_h, cv)

    zero = jnp.zeros((16,), jnp.float32)

    @pl.loop(0, NPI)
    def _(i):
        sl = pl.ds(i * 16, 16)
        esum_l[sl] = zero
        deg_l[sl] = zero

    cval = cv[...]
    ones = jnp.ones((16,), jnp.float32)

    @pl.loop(0, VI)
    def _(i):
        sl = pl.ds(i * 16, 16)
        s16 = src_c[sl]
        d16 = dst_c[sl]
        elv = plsc.load_gather(el_t, [s16])
        erv = plsc.load_gather(er_t, [d16])
        raw = elv + erv + ee_c[sl]
        e = jnp.where(raw >= 0, raw, NEG * raw)
        pv = jnp.exp(e - cval)
        p_c[sl] = pv
        plsc.addupdate_scatter(esum_l, [d16], pv)
        plsc.addupdate_scatter(deg_l, [d16], ones)

    pltpu.sync_copy(p_c, p_h.at[pl.ds(base, EW)])
    pltpu.sync_copy(esum_l, esum_h.at[pl.ds(wid * NP, NP)])
    pltpu.sync_copy(deg_l, deg_h.at[pl.ds(wid * NP, NP)])


def _mid_body(esum32_ref, deg32_ref, esum_ref, norm_ref):
    esum_ref[...] = jnp.sum(esum32_ref[...], axis=0)
    deg = jnp.sum(deg32_ref[...], axis=0)
    norm_ref[...] = 1.0 / jnp.maximum(deg, 1.0)


def _mid(esum32, deg32):
    return pl.pallas_call(
        _mid_body,
        out_shape=(
            jax.ShapeDtypeStruct((NP,), jnp.float32),
            jax.ShapeDtypeStruct((NP,), jnp.float32),
        ),
    )(esum32, deg32)


KB = 25
BLK = KB * BB
NBAT = NB


@functools.partial(
    pl.kernel,
    out_type=jax.ShapeDtypeStruct((2, NP, F), jnp.float32),
    mesh=_mesh,
    compiler_params=_sc_params,
    scratch_types=[
        pltpu.VMEM((BLK,), jnp.int32),
        pltpu.VMEM((BLK,), jnp.int32),
        pltpu.VMEM((BLK,), jnp.float32),
        pltpu.VMEM((NP,), jnp.float32),
        pltpu.VMEM((BB,), jnp.float32),
        pltpu.VMEM((BB,), jnp.int32),
        pltpu.VMEM((BB,), jnp.int32),
        pltpu.VMEM((BB,), jnp.int32),
        pltpu.VMEM((BB,), jnp.int32),
        pltpu.VMEM((BB, F), jnp.float32),
        pltpu.VMEM((BB, F), jnp.float32),
        pltpu.VMEM_SHARED((NP, F), jnp.float32),
        pltpu.SemaphoreType.DMA,
        pltpu.SemaphoreType.DMA,
        pltpu.SemaphoreType.DMA,
        pltpu.SemaphoreType.DMA,
    ],
)
def _phase2(src_h, dst_h, p_h, esum_h, feat_h, rst_h,
            src_blk, dst_blk, p_blk, es_t, a_buf,
            sidx0, sidx1, didx0, didx1, rows0, rows1, rst_s,
            gsem0, gsem1, ssem0, ssem1):
    c = lax.axis_index("c")
    s = lax.axis_index("s")
    wid = s * NC + c
    base = wid * EW
    pltpu.sync_copy(esum_h, es_t)

    def load_block(kb):
        off = base + kb * BLK
        pltpu.sync_copy(src_h.at[pl.ds(off, BLK)], src_blk)
        pltpu.sync_copy(dst_h.at[pl.ds(off, BLK)], dst_blk)
        pltpu.sync_copy(p_h.at[pl.ds(off, BLK)], p_blk)

    def fill_idx(sidx, didx, lb):
        @pl.loop(0, BB // 16)
        def _(j):
            dst_sl = pl.ds(j * 16, 16)
            src_sl = pl.ds(lb * BB + j * 16, 16)
            sidx[dst_sl] = src_blk[src_sl]
            didx[dst_sl] = dst_blk[src_sl]

    load_block(0)
    fill_idx(sidx0, didx0, 0)

    zero = jnp.zeros((16,), jnp.float32)

    @pl.loop(0, BB)
    def _(r):
        for j in range(F // 16):
            rows0[r, pl.ds(j * 16, 16)] = zero

    @pl.loop(0, RT // BB)
    def _(k):
        pltpu.sync_copy(rows0, rst_s.at[pl.ds(s * RT + k * BB, BB), :])

    plsc.subcore_barrier()
    pltpu.async_copy(feat_h.at[sidx0], rows0, gsem0)

    def body(i, rows_x, sidx_x, didx_x, gsem_x, ssem_x,
             rows_y, sidx_y, didx_y, gsem_y, ssem_y):
        pltpu.make_async_copy(feat_h.at[sidx_x], rows_x, gsem_x).wait()
        lb = lax.rem(i, KB)

        @pl.loop(0, BB // 16)
        def _(j):
            sl = pl.ds(lb * BB + j * 16, 16)
            d16 = dst_blk[sl]
            ev = plsc.load_gather(es_t, [d16])
            a_buf[pl.ds(j * 16, 16)] = p_blk[sl] / ev

        nb = i + 1

        @pl.when(nb < NBAT)
        def _():
            @pl.when(lax.rem(nb, KB) == 0)
            def _():
                load_block(nb // KB)

            @pl.when(i >= 1)
            def _():
                pltpu.make_async_copy(rows_y, rst_s.at[didx_y], ssem_y).wait()

            fill_idx(sidx_y, didx_y, lax.rem(nb, KB))
            pltpu.async_copy(feat_h.at[sidx_y], rows_y, gsem_y)

        @pl.loop(0, BB // 16)
        def _(g):
            a16 = a_buf[pl.ds(g * 16, 16)]
            for r in range(16):
                e = g * 16 + r
                av = a16[r]
                for j in range(F // 16):
                    sl = pl.ds(j * 16, 16)
                    rows_x[e, sl] = rows_x[e, sl] * av

        pltpu.async_copy(rows_x, rst_s.at[didx_x], ssem_x, add=True)

    slot0 = (rows0, sidx0, didx0, gsem0, ssem0)
    slot1 = (rows1, sidx1, didx1, gsem1, ssem1)

    @pl.loop(0, NBAT)
    def _(i):
        par = lax.rem(i, 2)

        @pl.when(par == 0)
        def _():
            body(i, *slot0, *slot1)

        @pl.when(par == 1)
        def _():
            body(i, *slot1, *slot0)

    pltpu.make_async_copy(rows1, rst_s.at[didx1], ssem1).wait()
    pltpu.make_async_copy(rows0, rst_s.at[didx0], ssem0).wait()

    plsc.subcore_barrier()
    pltpu.sync_copy(rst_s.at[pl.ds(s * RT, RT), :],
                    rst_h.at[c, pl.ds(s * RT, RT), :])


def _final_body(rst_ref, norm_ref, res_ref, o_ref):
    norm = norm_ref[...][:N]
    rst = rst_ref[0, :N, :] + rst_ref[1, :N, :]
    o_ref[...] = rst * norm[:, None] + res_ref[...]


def _final(rst_part, norm, resval):
    return pl.pallas_call(
        _final_body,
        out_shape=jax.ShapeDtypeStruct((N, F), jnp.float32),
    )(rst_part, norm, resval)


def kernel(x, edge_index, edge_attr, W_fc, W_edge, attn_l, attn_r, attn_edge,
           W_res):
    src = edge_index[0]
    dst = edge_index[1]
    al = attn_l.reshape(F)
    ar = attn_r.reshape(F)
    ae = attn_edge.reshape(F)
    ee, eemax = _escore(edge_attr.T, W_edge, ae)
    feat, resval, el, er, cvec = _dense(x, W_fc, W_res, al, ar, eemax)
    p, esum32, deg32 = _phase1(src, dst, el, er, ee, cvec)
    esum_f, norm_f = _mid(esum32.reshape(NW, NP), deg32.reshape(NW, NP))
    rst_part = _phase2(src, dst, p, esum_f, feat)
    out = _final(rst_part, norm_f, resval)
    return out.reshape(N, 1, F)
